# baseline (speedup 1.0000x reference)
import jax
import jax.numpy as jnp
from jax.experimental import pallas as pl
from jax.experimental.pallas import tpu as pltpu

_BB = 16
_SIGMA = (0, 4, 2, 6, 1, 5, 3, 7)


def _body(x_ref, o_ref):
    x = x_ref[...]
    C = x.shape[2]
    iota_i = jax.lax.broadcasted_iota(jnp.int32, (8, C), 0)
    iota_f = iota_i.astype(jnp.float32)
    row_c = [iota_f + 8.0 * k for k in range(4)]
    sent = jnp.full((8, C), 64.0, jnp.float32)
    m4 = iota_i < 4
    mq = (iota_i % 4) < 2
    me = (iota_i % 2) == 0
    for j in range(_BB // 8):
        slots = [None] * 8
        for t in range(8):
            xb = x[8 * j + t]
            g = [xb[8 * k:8 * (k + 1), :] for k in range(4)]
            t01 = jnp.minimum(g[0], g[1])
            t23 = jnp.minimum(g[2], g[3])
            v = jnp.minimum(t01, t23)
            for sh in (4, 2, 1):
                v = jnp.minimum(v, pltpu.roll(v, sh, axis=0))
            k01 = jnp.where(g[0] == v, row_c[0], row_c[1])
            k23 = jnp.where(g[2] == v, row_c[2], row_c[3])
            km = jnp.where(t23 == v, k23, sent)
            slots[_SIGMA[t]] = jnp.where(t01 == v, k01, km)
        c = []
        for p in range(4):
            a = jnp.where(m4, slots[2 * p], slots[2 * p + 1])
            bsw = jnp.where(m4, slots[2 * p + 1], slots[2 * p])
            c.append(jnp.minimum(a, pltpu.roll(bsw, 4, axis=0)))
        d = []
        for p in range(2):
            e = jnp.minimum(c[2 * p], pltpu.roll(c[2 * p], 6, axis=0))
            f = jnp.minimum(c[2 * p + 1], pltpu.roll(c[2 * p + 1], 2, axis=0))
            d.append(jnp.where(mq, e, f))
        e = jnp.minimum(d[0], pltpu.roll(d[0], 7, axis=0))
        f = jnp.minimum(d[1], pltpu.roll(d[1], 1, axis=0))
        o_ref[8 * j:8 * (j + 1), :] = jnp.where(me, e, f).astype(jnp.int32)


def kernel(x):
    B, R, C = x.shape
    return pl.pallas_call(
        _body,
        grid=(B // _BB,),
        in_specs=[pl.BlockSpec((_BB, R, C), lambda i: (i, 0, 0))],
        out_specs=pl.BlockSpec((_BB, C), lambda i: (i, 0)),
        out_shape=jax.ShapeDtypeStruct((B, C), jnp.int32),
    )(x)


# speedup vs baseline: 1.0245x; 1.0245x over previous
import jax
import jax.numpy as jnp
from jax.experimental import pallas as pl
from jax.experimental.pallas import tpu as pltpu

_BB = 8
_SIGMA = (0, 4, 2, 6, 1, 5, 3, 7)


def _body(x_ref, o_ref):
    x = x_ref[...]
    C = x.shape[2]
    iota_i = jax.lax.broadcasted_iota(jnp.int32, (8, C), 0)
    iota_f = iota_i.astype(jnp.float32)
    row_c = [iota_f + 8.0 * k for k in range(4)]
    sent = jnp.full((8, C), 64.0, jnp.float32)
    m4 = iota_i < 4
    mq = (iota_i % 4) < 2
    me = (iota_i % 2) == 0
    for j in range(_BB // 8):
        slots = [None] * 8
        for t in range(8):
            xb = x[8 * j + t]
            g = [xb[8 * k:8 * (k + 1), :] for k in range(4)]
            t01 = jnp.minimum(g[0], g[1])
            t23 = jnp.minimum(g[2], g[3])
            v = jnp.minimum(t01, t23)
            for sh in (4, 2, 1):
                v = jnp.minimum(v, pltpu.roll(v, sh, axis=0))
            k01 = jnp.where(g[0] == v, row_c[0], row_c[1])
            k23 = jnp.where(g[2] == v, row_c[2], row_c[3])
            km = jnp.where(t23 == v, k23, sent)
            slots[_SIGMA[t]] = jnp.where(t01 == v, k01, km)
        c = []
        for p in range(4):
            a = jnp.where(m4, slots[2 * p], slots[2 * p + 1])
            bsw = jnp.where(m4, slots[2 * p + 1], slots[2 * p])
            c.append(jnp.minimum(a, pltpu.roll(bsw, 4, axis=0)))
        d = []
        for p in range(2):
            e = jnp.minimum(c[2 * p], pltpu.roll(c[2 * p], 6, axis=0))
            f = jnp.minimum(c[2 * p + 1], pltpu.roll(c[2 * p + 1], 2, axis=0))
            d.append(jnp.where(mq, e, f))
        e = jnp.minimum(d[0], pltpu.roll(d[0], 7, axis=0))
        f = jnp.minimum(d[1], pltpu.roll(d[1], 1, axis=0))
        o_ref[8 * j:8 * (j + 1), :] = jnp.where(me, e, f).astype(jnp.int32)


def kernel(x):
    B, R, C = x.shape
    return pl.pallas_call(
        _body,
        grid=(B // _BB,),
        in_specs=[pl.BlockSpec((_BB, R, C), lambda i: (i, 0, 0))],
        out_specs=pl.BlockSpec((_BB, C), lambda i: (i, 0)),
        out_shape=jax.ShapeDtypeStruct((B, C), jnp.int32),
    )(x)
